# baseline (speedup 1.0000x reference)
import jax
import jax.numpy as jnp
from jax import lax
from jax.experimental import pallas as pl

_CH = 62
_CHP = 64
_TOPK = 10


def _mm(a, b):
    return jnp.dot(a, b, preferred_element_type=jnp.float32)


def _bmm(a, b):
    return lax.dot_general(a, b, (((2,), (1,)), ((0,), (0,))),
                           preferred_element_type=jnp.float32)


def _body(refs, bb):
    (x_ref, Wg, bg, Wb, bbias,
     Wr1, br1, Wt1, Wr2, br2, Wt2, Wr3, br3, Wt3, Wr4, br4, Wt4,
     Wr5, br5, Wt5, Wr6, br6, Wt6, Wr7, br7, Wt7, o_ref) = refs

    xv = x_ref[...]
    xp = jnp.concatenate(
        [xv, jnp.zeros((bb, _CHP - _CH, xv.shape[2]), jnp.float32)], axis=1)
    x2 = xp.reshape(bb * _CHP, xv.shape[2])

    x16 = x2.astype(jnp.bfloat16)
    xm = jnp.tanh(_mm(x16, Wg[...]) + bg[...])
    xa = jnp.tanh(_mm(x2, Wb[...]) + bbias[...])

    xap = xa.reshape(bb // 2, 2 * _CHP, xa.shape[1])
    bd = lax.dot_general(xap, xap, (((2,), (2,)), ((0,), (0,))),
                         preferred_element_type=jnp.float32)

    lane = lax.broadcasted_iota(jnp.int32, (1, 1, 2 * _CHP), 2)
    E = jnp.where(lane < _CHP, bd[:, :_CHP, :], bd[:, _CHP:, :])
    jrow = lax.broadcasted_iota(jnp.int32, (1, _CHP, 1), 1)
    e = jnp.where(jrow < _CH, jnp.exp(E - 64.0), 0.0)
    S = jnp.sum(e, axis=1, keepdims=True)

    work = e
    for _ in range(_TOPK):
        m = jnp.max(work, axis=1, keepdims=True)
        m = jnp.where(m >= 0.0, m, 100.0)
        work = jnp.where(work == m, work - 2.0, work)
    Gp = jnp.where(work < 0.0, e, 0.0) / S

    Gbd = jnp.concatenate([jnp.where(lane < _CHP, Gp, 0.0),
                           jnp.where(lane >= _CHP, Gp, 0.0)], axis=1)
    Gbd16 = Gbd.astype(jnp.bfloat16)

    def gconv(h16, Wr, br, Wt):
        f = h16.shape[1]
        hp = h16.reshape(bb // 2, 2 * _CHP, f)
        a3 = lax.dot_general(Gbd16, hp, (((1,), (1,)), ((0,), (0,))),
                             preferred_element_type=jnp.float32)
        a2 = a3.reshape(bb * _CHP, f).astype(jnp.bfloat16)
        return _mm(a2, Wr[...]) + br[...] + _mm(h16, Wt[...])

    h = jax.nn.relu(gconv(x16, Wr1, br1, Wt1))
    h = h + jax.nn.relu(gconv(h.astype(jnp.bfloat16), Wr2, br2, Wt2))
    h = h + jax.nn.relu(gconv(h.astype(jnp.bfloat16), Wr3, br3, Wt3))
    h = h + jax.nn.relu(gconv(h.astype(jnp.bfloat16), Wr4, br4, Wt4))
    h = h + jax.nn.relu(gconv(h.astype(jnp.bfloat16), Wr5, br5, Wt5))
    h = h + jax.nn.relu(gconv(h.astype(jnp.bfloat16), Wr6, br6, Wt6))
    h = jax.nn.relu(gconv(h.astype(jnp.bfloat16), Wr7, br7, Wt7))

    out = (h * xm).reshape(bb, _CHP, xm.shape[1])
    o_ref[...] = out[:, :_CH, :]


def kernel(x, Wg, bg, Wb, bb, Wr1, br1, Wt1, Wr2, br2, Wt2, Wr3, br3, Wt3,
           Wr4, br4, Wt4, Wr5, br5, Wt5, Wr6, br6, Wt6, Wr7, br7, Wt7):
    B = x.shape[0]
    BB = 128
    while B % BB:
        BB //= 2
    grid = (B // BB,)

    bf = jnp.bfloat16
    weights = [Wg.astype(bf), bg.reshape(1, -1), Wb, bb.reshape(1, -1)]
    for Wr, br, Wt in ((Wr1, br1, Wt1), (Wr2, br2, Wt2), (Wr3, br3, Wt3),
                       (Wr4, br4, Wt4), (Wr5, br5, Wt5), (Wr6, br6, Wt6),
                       (Wr7, br7, Wt7)):
        weights += [Wr.astype(bf), br.reshape(1, -1), Wt.astype(bf)]

    in_specs = [pl.BlockSpec((BB, _CH, x.shape[2]), lambda i: (i, 0, 0))]
    for w in weights:
        in_specs.append(pl.BlockSpec(w.shape, lambda i: (0,) * w.ndim))

    out_f = Wr7.shape[1]
    fn = lambda *refs: _body(refs, BB)
    return pl.pallas_call(
        fn,
        grid=grid,
        in_specs=in_specs,
        out_specs=pl.BlockSpec((BB, _CH, out_f), lambda i: (i, 0, 0)),
        out_shape=jax.ShapeDtypeStruct((B, _CH, out_f), jnp.float32),
    )(x, *weights)


# speedup vs baseline: 1.3903x; 1.3903x over previous
import jax
import jax.numpy as jnp
from jax import lax
from jax.experimental import pallas as pl

_CH = 62
_CHP = 64
_TOPK = 10


def _mm(a, b):
    return jnp.dot(a, b, preferred_element_type=jnp.float32)


def _bmm(a, b):
    return lax.dot_general(a, b, (((2,), (1,)), ((0,), (0,))),
                           preferred_element_type=jnp.float32)


def _body(refs, bb):
    (x_ref, Wg, bg, Wb, bbias,
     Wrt1, br1, Wrt2, br2, Wrt3, br3, Wrt4, br4,
     Wrt5, br5, Wrt6, br6, Wrt7, br7, o_ref) = refs

    xv = x_ref[...]
    xp = jnp.concatenate(
        [xv, jnp.zeros((bb, _CHP - _CH, xv.shape[2]), jnp.float32)], axis=1)
    x2 = xp.reshape(bb * _CHP, xv.shape[2])

    xm = jnp.tanh(_mm(x2, Wg[...]) + bg[...])
    xa = jnp.tanh(_mm(x2, Wb[...]) + bbias[...])

    xap = xa.reshape(bb // 2, 2 * _CHP, xa.shape[1])
    bd = lax.dot_general(xap, xap, (((2,), (2,)), ((0,), (0,))),
                         preferred_element_type=jnp.float32)

    lane = lax.broadcasted_iota(jnp.int32, (1, 1, 2 * _CHP), 2)
    E = jnp.where(lane < _CHP, bd[:, :_CHP, :], bd[:, _CHP:, :])
    jrow = lax.broadcasted_iota(jnp.int32, (1, _CHP, 1), 1)
    e = jnp.where(jrow < _CH, jnp.exp(E - 64.0), 0.0)
    S = jnp.sum(e, axis=1, keepdims=True)

    work = e
    for _ in range(_TOPK):
        m = jnp.max(work, axis=1, keepdims=True)
        work = jnp.where(work == m, work - 2.0, work)
    Gp = jnp.where(work < 0.0, e, 0.0) / S

    Gbd = jnp.concatenate([jnp.where(lane < _CHP, Gp, 0.0),
                           jnp.where(lane >= _CHP, Gp, 0.0)], axis=1)
    def gconv(h2, Wrt, br):
        u2 = _mm(h2, Wrt[...])
        fo = u2.shape[1] // 2
        uR = u2[:, :fo].reshape(bb // 2, 2 * _CHP, fo)
        a3 = lax.dot_general(Gbd, uR, (((1,), (1,)), ((0,), (0,))),
                             preferred_element_type=jnp.float32)
        return a3.reshape(bb * _CHP, fo) + u2[:, fo:] + br[...]

    h = jax.nn.relu(gconv(x2, Wrt1, br1))
    h = h + jax.nn.relu(gconv(h, Wrt2, br2))
    h = h + jax.nn.relu(gconv(h, Wrt3, br3))
    h = h + jax.nn.relu(gconv(h, Wrt4, br4))
    h = h + jax.nn.relu(gconv(h, Wrt5, br5))
    h = h + jax.nn.relu(gconv(h, Wrt6, br6))
    h = jax.nn.relu(gconv(h, Wrt7, br7))

    out = (h * xm).reshape(bb, _CHP, xm.shape[1])
    o_ref[...] = out[:, :_CH, :]


def kernel(x, Wg, bg, Wb, bb, Wr1, br1, Wt1, Wr2, br2, Wt2, Wr3, br3, Wt3,
           Wr4, br4, Wt4, Wr5, br5, Wt5, Wr6, br6, Wt6, Wr7, br7, Wt7):
    B = x.shape[0]
    BB = 128
    while B % BB:
        BB //= 2
    grid = (B // BB,)

    weights = [Wg, bg.reshape(1, -1), Wb, bb.reshape(1, -1)]
    for Wr, br, Wt in ((Wr1, br1, Wt1), (Wr2, br2, Wt2), (Wr3, br3, Wt3),
                       (Wr4, br4, Wt4), (Wr5, br5, Wt5), (Wr6, br6, Wt6),
                       (Wr7, br7, Wt7)):
        weights += [jnp.concatenate([Wr, Wt], axis=1), br.reshape(1, -1)]

    in_specs = [pl.BlockSpec((BB, _CH, x.shape[2]), lambda i: (i, 0, 0))]
    for w in weights:
        in_specs.append(pl.BlockSpec(w.shape, lambda i: (0,) * w.ndim))

    out_f = Wr7.shape[1]
    fn = lambda *refs: _body(refs, BB)
    return pl.pallas_call(
        fn,
        grid=grid,
        in_specs=in_specs,
        out_specs=pl.BlockSpec((BB, _CH, out_f), lambda i: (i, 0, 0)),
        out_shape=jax.ShapeDtypeStruct((B, _CH, out_f), jnp.float32),
    )(x, *weights)
